# baseline (speedup 1.0000x reference)
import jax
import jax.numpy as jnp
from jax import lax
from jax.experimental import pallas as pl
from jax.experimental.pallas import tpu as pltpu
from jax.experimental.pallas import tpu_sc as plsc

N = 10000
E = 320000
D = 128
NPAD = 10240
NW = 32
EPT = 10240
EPAD = NW * EPT
DCHUNK = 128
NDCHUNK = EPT // DCHUNK
GCHUNK = 64
NCHUNK = EPT // GCHUNK
NBUF = 2
GRP = 16
NSUB = 16
ROWS_PER_SUB = NPAD // NSUB
DEG_W = 16


def _sc_deg_body(dst_hbm, out_hbm, didx_v, ones_v, zb_v, acc_s):
    cid = lax.axis_index("c")
    sid = lax.axis_index("s")
    wid = sid * 2 + cid

    @pl.loop(0, DCHUNK)
    def _(r):
        ones_v[r, :] = jnp.ones((DEG_W,), jnp.float32)
        zb_v[r, :] = jnp.zeros((DEG_W,), jnp.float32)

    row0 = sid * ROWS_PER_SUB
    for k in range(ROWS_PER_SUB // DCHUNK):
        pltpu.sync_copy(zb_v, acc_s.at[pl.ds(row0 + k * DCHUNK, DCHUNK)])
    pltpu.sync_copy(dst_hbm.at[pl.ds(wid * NDCHUNK, NDCHUNK)], didx_v)
    plsc.subcore_barrier()

    @pl.loop(0, NDCHUNK)
    def _(ci):
        pltpu.sync_copy(ones_v, acc_s.at[didx_v.at[ci]], add=True)

    plsc.subcore_barrier()
    pltpu.sync_copy(acc_s.at[pl.ds(row0, ROWS_PER_SUB)],
                    out_hbm.at[cid, pl.ds(row0, ROWS_PER_SUB)])


def _sc_deg(dst2):
    mesh = plsc.VectorSubcoreMesh(core_axis_name="c", subcore_axis_name="s")
    f = pl.kernel(
        _sc_deg_body,
        out_type=jax.ShapeDtypeStruct((2, NPAD, DEG_W), jnp.float32),
        mesh=mesh,
        scratch_types=[
            pltpu.VMEM((NDCHUNK, DCHUNK), jnp.int32),
            pltpu.VMEM((DCHUNK, DEG_W), jnp.float32),
            pltpu.VMEM((DCHUNK, DEG_W), jnp.float32),
            pltpu.VMEM_SHARED((NPAD, DEG_W), jnp.float32),
        ],
    )
    return f(dst2)


def _sc_agg_body(g_hbm, src_hbm, dst_hbm, out_hbm, sidx_v, didx_v,
                 buf0, buf1, acc_s, sem0, sem1):
    cid = lax.axis_index("c")
    sid = lax.axis_index("s")
    wid = sid * 2 + cid
    bufs = (buf0, buf1)
    sems = (sem0, sem1)

    def src_idx(ci):
        return sidx_v.at[ci // 2, pl.ds((ci % 2) * GCHUNK, GCHUNK)]

    @pl.loop(0, GCHUNK)
    def _(r):
        for c in range(D // 16):
            buf0[r, pl.ds(c * 16, 16)] = jnp.zeros((16,), jnp.float32)

    row0 = sid * ROWS_PER_SUB
    for k in range(ROWS_PER_SUB // GCHUNK):
        pltpu.sync_copy(buf0, acc_s.at[pl.ds(row0 + k * GCHUNK, GCHUNK)])
    pltpu.sync_copy(src_hbm.at[pl.ds(wid * (NCHUNK // 2), NCHUNK // 2)],
                    sidx_v)
    pltpu.sync_copy(dst_hbm.at[pl.ds(wid * NCHUNK, NCHUNK)], didx_v)

    plsc.subcore_barrier()

    @pl.loop(0, NCHUNK)
    def _(ci):
        pltpu.async_copy(g_hbm.at[src_idx(ci)], buf0, sem0).wait()
        pltpu.sync_copy(buf0, acc_s.at[didx_v.at[ci]], add=True)

    plsc.subcore_barrier()
    pltpu.sync_copy(acc_s.at[pl.ds(row0, ROWS_PER_SUB)],
                    out_hbm.at[cid, pl.ds(row0, ROWS_PER_SUB)])


def _sc_agg(g, src2, dst2):
    mesh = plsc.VectorSubcoreMesh(core_axis_name="c", subcore_axis_name="s")
    f = pl.kernel(
        _sc_agg_body,
        out_type=jax.ShapeDtypeStruct((2, NPAD, D), jnp.float32),
        mesh=mesh,
        scratch_types=[
            pltpu.VMEM((NCHUNK // 2, 2 * GCHUNK), jnp.int32),
            pltpu.VMEM((NCHUNK, GCHUNK), jnp.int32),
            pltpu.VMEM((GCHUNK, D), jnp.float32),
            pltpu.VMEM((GCHUNK, D), jnp.float32),
            pltpu.VMEM_SHARED((NPAD, D), jnp.float32),
            pltpu.SemaphoreType.DMA,
            pltpu.SemaphoreType.DMA,
        ],
    )
    return f(g, src2, dst2)


def _d_from_degp(degp_ref):
    deg = degp_ref[0, :, 0:1] + degp_ref[1, :, 0:1] + 1.0
    return lax.rsqrt(deg)


def _tc_first_body(x_ref, degp_ref, w_ref, o_ref):
    d = _d_from_degp(degp_ref)
    h = jnp.dot(x_ref[...], w_ref[...],
                precision=lax.Precision.HIGHEST,
                preferred_element_type=jnp.float32)
    o_ref[...] = d * h


def _tc_first(xp, degp, W1):
    return pl.pallas_call(
        _tc_first_body,
        out_shape=jax.ShapeDtypeStruct((NPAD, D), jnp.float32),
    )(xp, degp, W1)


def _tc_mid_body(q_ref, g1_ref, degp_ref, b1_ref, w2_ref, o_ref):
    d = _d_from_degp(degp_ref)
    s = q_ref[0] + q_ref[1] + g1_ref[...]
    z = jnp.maximum(d * s + b1_ref[...], 0.0)
    h = jnp.dot(z, w2_ref[...],
                precision=lax.Precision.HIGHEST,
                preferred_element_type=jnp.float32)
    o_ref[...] = d * h


def _tc_mid(q, g1, degp, b1, W2):
    return pl.pallas_call(
        _tc_mid_body,
        out_shape=jax.ShapeDtypeStruct((NPAD, D), jnp.float32),
    )(q, g1, degp, b1, W2)


def _tc_head_body(r_ref, g2_ref, degp_ref, b2_ref, wfc_ref, bfc_ref, o_ref):
    d = _d_from_degp(degp_ref)
    s = r_ref[0] + r_ref[1] + g2_ref[...]
    z2 = jnp.maximum(d * s + b2_ref[...], 0.0)
    h = jnp.dot(z2, wfc_ref[...],
                precision=lax.Precision.HIGHEST,
                preferred_element_type=jnp.float32)
    o_ref[...] = jax.nn.sigmoid(h + bfc_ref[...])


def _tc_head(r, g2, degp, b2, Wfcp, bfcp):
    return pl.pallas_call(
        _tc_head_body,
        out_shape=jax.ShapeDtypeStruct((NPAD, 8), jnp.float32),
    )(r, g2, degp, b2, Wfcp, bfcp)


def kernel(x, edge_index, W1, b1, W2, b2, Wfc, bfc):
    xp = jnp.concatenate([x, jnp.zeros((NPAD - N, D), x.dtype)], axis=0)
    pad_idx = jnp.full((EPAD - E,), N, jnp.int32)
    src_pad = jnp.concatenate([edge_index[0], pad_idx])
    dst_pad = jnp.concatenate([edge_index[1], pad_idx])
    src2 = src_pad.reshape(NW * (NCHUNK // 2), 2 * GCHUNK)
    dst2 = dst_pad.reshape(NW * NCHUNK, GCHUNK)
    dst2d = dst_pad.reshape(NW * NDCHUNK, DCHUNK)

    degp = _sc_deg(dst2d)
    g1 = _tc_first(xp, degp, W1)
    q = _sc_agg(g1, src2, dst2)
    g2 = _tc_mid(q, g1, degp, b1.reshape(1, D), W2)
    r = _sc_agg(g2, src2, dst2)
    Wfcp = jnp.pad(Wfc, ((0, 0), (0, 7)))
    bfcp = jnp.broadcast_to(bfc.reshape(1, 1), (1, 8))
    ph = _tc_head(r, g2, degp, b2.reshape(1, D), Wfcp, bfcp)
    return ph[:N, 0:1]


# speedup vs baseline: 2.2025x; 2.2025x over previous
import jax
import jax.numpy as jnp
from jax import lax
from jax.experimental import pallas as pl
from jax.experimental.pallas import tpu as pltpu
from jax.experimental.pallas import tpu_sc as plsc

N = 10000
E = 320000
D = 128
HALF = 64
NPAD = 10240
NW = 32
EPT = 10240
EPAD = NW * EPT
CHUNK = 128
NCHUNK = EPT // CHUNK
NSUB = 16
ROWS_PER_SUB = NPAD // NSUB
DEG_W = 16


def _sc_deg_body(dst_hbm, out_hbm, didx_v, ones_v, zb_v, acc_s):
    cid = lax.axis_index("c")
    sid = lax.axis_index("s")
    wid = sid * 2 + cid

    @pl.loop(0, CHUNK)
    def _(r):
        ones_v[r, :] = jnp.ones((DEG_W,), jnp.float32)
        zb_v[r, :] = jnp.zeros((DEG_W,), jnp.float32)

    row0 = sid * ROWS_PER_SUB
    for k in range(ROWS_PER_SUB // CHUNK):
        pltpu.sync_copy(zb_v, acc_s.at[pl.ds(row0 + k * CHUNK, CHUNK)])
    pltpu.sync_copy(dst_hbm.at[pl.ds(wid * NCHUNK, NCHUNK)], didx_v)
    plsc.subcore_barrier()

    @pl.loop(0, NCHUNK)
    def _(ci):
        pltpu.sync_copy(ones_v, acc_s.at[didx_v.at[ci]], add=True)

    plsc.subcore_barrier()
    pltpu.sync_copy(acc_s.at[pl.ds(row0, ROWS_PER_SUB)],
                    out_hbm.at[cid, pl.ds(row0, ROWS_PER_SUB)])


def _sc_deg(dst2):
    mesh = plsc.VectorSubcoreMesh(core_axis_name="c", subcore_axis_name="s")
    f = pl.kernel(
        _sc_deg_body,
        out_type=jax.ShapeDtypeStruct((2, NPAD, DEG_W), jnp.float32),
        mesh=mesh,
        scratch_types=[
            pltpu.VMEM((NCHUNK, CHUNK), jnp.int32),
            pltpu.VMEM((CHUNK, DEG_W), jnp.float32),
            pltpu.VMEM((CHUNK, DEG_W), jnp.float32),
            pltpu.VMEM_SHARED((NPAD, DEG_W), jnp.float32),
        ],
    )
    return f(dst2)


def _zero_buf(buf_v):
    @pl.loop(0, CHUNK)
    def _(r):
        for c in range(HALF // 16):
            buf_v[r, pl.ds(c * 16, 16)] = jnp.zeros((16,), jnp.float32)


def _sc_agg_body(g2h_hbm, src_hbm, dst_hbm, out_hbm, sidx_v, didx_v, buf_v,
                 tab_s, acc_s, sem):
    cid = lax.axis_index("c")
    sid = lax.axis_index("s")
    wid = sid * 2 + cid
    row0 = sid * ROWS_PER_SUB

    pltpu.sync_copy(src_hbm.at[pl.ds(wid * NCHUNK, NCHUNK)], sidx_v)
    pltpu.sync_copy(dst_hbm.at[pl.ds(wid * NCHUNK, NCHUNK)], didx_v)

    for h in range(2):
        _zero_buf(buf_v)
        pltpu.sync_copy(g2h_hbm.at[h, pl.ds(row0, ROWS_PER_SUB)],
                        tab_s.at[pl.ds(row0, ROWS_PER_SUB)])
        for k in range(ROWS_PER_SUB // CHUNK):
            pltpu.sync_copy(buf_v, acc_s.at[pl.ds(row0 + k * CHUNK, CHUNK)])
        plsc.subcore_barrier()

        @pl.loop(0, NCHUNK)
        def _(ci):
            pltpu.async_copy(tab_s.at[sidx_v.at[ci]], buf_v, sem).wait()
            pltpu.sync_copy(buf_v, acc_s.at[didx_v.at[ci]], add=True)

        plsc.subcore_barrier()
        pltpu.sync_copy(acc_s.at[pl.ds(row0, ROWS_PER_SUB)],
                        out_hbm.at[cid, h, pl.ds(row0, ROWS_PER_SUB)])


def _sc_agg(g2h, src2, dst2):
    mesh = plsc.VectorSubcoreMesh(core_axis_name="c", subcore_axis_name="s")
    f = pl.kernel(
        _sc_agg_body,
        out_type=jax.ShapeDtypeStruct((2, 2, NPAD, HALF), jnp.float32),
        mesh=mesh,
        compiler_params=pltpu.CompilerParams(use_tc_tiling_on_sc=False),
        scratch_types=[
            pltpu.VMEM((NCHUNK, CHUNK), jnp.int32),
            pltpu.VMEM((NCHUNK, CHUNK), jnp.int32),
            pltpu.VMEM((CHUNK, HALF), jnp.float32),
            pltpu.VMEM_SHARED((NPAD, HALF), jnp.float32),
            pltpu.VMEM_SHARED((NPAD, HALF), jnp.float32),
            pltpu.SemaphoreType.DMA,
        ],
    )
    return f(g2h, src2, dst2)


def _d_from_degp(degp_ref):
    deg = degp_ref[0, :, 0:1] + degp_ref[1, :, 0:1] + 1.0
    return lax.rsqrt(deg)


def _tc_first_body(x_ref, degp_ref, w_ref, o_ref):
    d = _d_from_degp(degp_ref)
    h = jnp.dot(x_ref[...], w_ref[...],
                precision=lax.Precision.HIGHEST,
                preferred_element_type=jnp.float32)
    g = d * h
    o_ref[0] = g[:, :HALF]
    o_ref[1] = g[:, HALF:]


BLK = 1024
NBLK = NPAD // BLK


def _tc_first(xp, degp, W1):
    return pl.pallas_call(
        _tc_first_body,
        grid=(NBLK,),
        in_specs=[
            pl.BlockSpec((BLK, D), lambda i: (i, 0)),
            pl.BlockSpec((2, BLK, DEG_W), lambda i: (0, i, 0)),
            pl.BlockSpec((D, D), lambda i: (0, 0)),
        ],
        out_specs=pl.BlockSpec((2, BLK, HALF), lambda i: (0, i, 0)),
        out_shape=jax.ShapeDtypeStruct((2, NPAD, HALF), jnp.float32),
    )(xp, degp, W1)


def _tc_mid_body(q_ref, g1_ref, degp_ref, b1_ref, w2_ref, o_ref):
    d = _d_from_degp(degp_ref)
    s_lo = q_ref[0, 0] + q_ref[1, 0] + g1_ref[0]
    s_hi = q_ref[0, 1] + q_ref[1, 1] + g1_ref[1]
    s = jnp.concatenate([s_lo, s_hi], axis=1)
    z = jnp.maximum(d * s + b1_ref[...], 0.0)
    h = jnp.dot(z, w2_ref[...],
                precision=lax.Precision.HIGHEST,
                preferred_element_type=jnp.float32)
    g = d * h
    o_ref[0] = g[:, :HALF]
    o_ref[1] = g[:, HALF:]


def _tc_mid(q, g1, degp, b1, W2):
    return pl.pallas_call(
        _tc_mid_body,
        grid=(NBLK,),
        in_specs=[
            pl.BlockSpec((2, 2, BLK, HALF), lambda i: (0, 0, i, 0)),
            pl.BlockSpec((2, BLK, HALF), lambda i: (0, i, 0)),
            pl.BlockSpec((2, BLK, DEG_W), lambda i: (0, i, 0)),
            pl.BlockSpec((1, D), lambda i: (0, 0)),
            pl.BlockSpec((D, D), lambda i: (0, 0)),
        ],
        out_specs=pl.BlockSpec((2, BLK, HALF), lambda i: (0, i, 0)),
        out_shape=jax.ShapeDtypeStruct((2, NPAD, HALF), jnp.float32),
    )(q, g1, degp, b1, W2)


def _tc_head_body(r_ref, g2_ref, degp_ref, b2_ref, wfc_ref, bfc_ref, o_ref):
    d = _d_from_degp(degp_ref)
    s_lo = r_ref[0, 0] + r_ref[1, 0] + g2_ref[0]
    s_hi = r_ref[0, 1] + r_ref[1, 1] + g2_ref[1]
    s = jnp.concatenate([s_lo, s_hi], axis=1)
    z2 = jnp.maximum(d * s + b2_ref[...], 0.0)
    h = jnp.dot(z2, wfc_ref[...],
                precision=lax.Precision.HIGHEST,
                preferred_element_type=jnp.float32)
    o_ref[...] = jax.nn.sigmoid(h + bfc_ref[...])


def _tc_head(r, g2, degp, b2, Wfcp, bfcp):
    return pl.pallas_call(
        _tc_head_body,
        grid=(NBLK,),
        in_specs=[
            pl.BlockSpec((2, 2, BLK, HALF), lambda i: (0, 0, i, 0)),
            pl.BlockSpec((2, BLK, HALF), lambda i: (0, i, 0)),
            pl.BlockSpec((2, BLK, DEG_W), lambda i: (0, i, 0)),
            pl.BlockSpec((1, D), lambda i: (0, 0)),
            pl.BlockSpec((D, 8), lambda i: (0, 0)),
            pl.BlockSpec((1, 8), lambda i: (0, 0)),
        ],
        out_specs=pl.BlockSpec((BLK, 8), lambda i: (i, 0)),
        out_shape=jax.ShapeDtypeStruct((NPAD, 8), jnp.float32),
    )(r, g2, degp, b2, Wfcp, bfcp)


def kernel(x, edge_index, W1, b1, W2, b2, Wfc, bfc):
    xp = jnp.concatenate([x, jnp.zeros((NPAD - N, D), x.dtype)], axis=0)
    pad_idx = jnp.full((EPAD - E,), N, jnp.int32)
    src_pad = jnp.concatenate([edge_index[0], pad_idx])
    dst_pad = jnp.concatenate([edge_index[1], pad_idx])
    src2 = src_pad.reshape(NW * NCHUNK, CHUNK)
    dst2 = dst_pad.reshape(NW * NCHUNK, CHUNK)

    degp = _sc_deg(dst2)
    g1 = _tc_first(xp, degp, W1)
    q = _sc_agg(g1, src2, dst2)
    g2 = _tc_mid(q, g1, degp, b1.reshape(1, D), W2)
    r = _sc_agg(g2, src2, dst2)
    Wfcp = jnp.pad(Wfc, ((0, 0), (0, 7)))
    bfcp = jnp.broadcast_to(bfc.reshape(1, 1), (1, 8))
    ph = _tc_head(r, g2, degp, b2.reshape(1, D), Wfcp, bfcp)
    return ph[:N, 0:1]


# speedup vs baseline: 2.6030x; 1.1818x over previous
import jax
import jax.numpy as jnp
from jax import lax
from jax.experimental import pallas as pl
from jax.experimental.pallas import tpu as pltpu
from jax.experimental.pallas import tpu_sc as plsc

N = 10000
E = 320000
D = 128
HALF = 64
NPAD = 10240
NW = 32
EPT = 10240
EPAD = NW * EPT
CHUNK = 128
NCHUNK = EPT // CHUNK
NSUB = 16
ROWS_PER_SUB = NPAD // NSUB
DEG_W = 16


def _sc_deg_body(dst_hbm, out_hbm, didx_v, ones_v, zb_v, acc_s):
    cid = lax.axis_index("c")
    sid = lax.axis_index("s")
    wid = sid * 2 + cid

    @pl.loop(0, CHUNK)
    def _(r):
        ones_v[r, :] = jnp.ones((DEG_W,), jnp.float32)
        zb_v[r, :] = jnp.zeros((DEG_W,), jnp.float32)

    row0 = sid * ROWS_PER_SUB
    for k in range(ROWS_PER_SUB // CHUNK):
        pltpu.sync_copy(zb_v, acc_s.at[pl.ds(row0 + k * CHUNK, CHUNK)])
    pltpu.sync_copy(dst_hbm.at[pl.ds(wid * NCHUNK, NCHUNK)], didx_v)
    plsc.subcore_barrier()

    @pl.loop(0, NCHUNK)
    def _(ci):
        pltpu.sync_copy(ones_v, acc_s.at[didx_v.at[ci]], add=True)

    plsc.subcore_barrier()
    pltpu.sync_copy(acc_s.at[pl.ds(row0, ROWS_PER_SUB)],
                    out_hbm.at[cid, pl.ds(row0, ROWS_PER_SUB)])


def _sc_deg(dst2):
    mesh = plsc.VectorSubcoreMesh(core_axis_name="c", subcore_axis_name="s")
    f = pl.kernel(
        _sc_deg_body,
        out_type=jax.ShapeDtypeStruct((2, NPAD, DEG_W), jnp.float32),
        mesh=mesh,
        scratch_types=[
            pltpu.VMEM((NCHUNK, CHUNK), jnp.int32),
            pltpu.VMEM((CHUNK, DEG_W), jnp.float32),
            pltpu.VMEM((CHUNK, DEG_W), jnp.float32),
            pltpu.VMEM_SHARED((NPAD, DEG_W), jnp.float32),
        ],
    )
    return f(dst2)


def _zero_buf(buf_v):
    @pl.loop(0, CHUNK)
    def _(r):
        for c in range(HALF // 16):
            buf_v[r, pl.ds(c * 16, 16)] = jnp.zeros((16,), jnp.float32)


def _sc_agg_body(g2h_hbm, src_hbm, dst_hbm, out_hbm, sidx_v, didx_v, buf_v,
                 buf2_v, tab_s, acc_s, sem, sem2):
    cid = lax.axis_index("c")
    sid = lax.axis_index("s")
    wid = sid * 2 + cid
    row0 = sid * ROWS_PER_SUB

    pltpu.sync_copy(src_hbm.at[pl.ds(wid * NCHUNK, NCHUNK)], sidx_v)
    pltpu.sync_copy(dst_hbm.at[pl.ds(wid * NCHUNK, NCHUNK)], didx_v)

    for h in range(2):
        _zero_buf(buf_v)
        pltpu.sync_copy(g2h_hbm.at[h, pl.ds(row0, ROWS_PER_SUB)],
                        tab_s.at[pl.ds(row0, ROWS_PER_SUB)])
        for k in range(ROWS_PER_SUB // CHUNK):
            pltpu.sync_copy(buf_v, acc_s.at[pl.ds(row0 + k * CHUNK, CHUNK)])
        plsc.subcore_barrier()

        @pl.loop(0, NCHUNK, step=8)
        def _(g0):
            bufs = (buf_v, buf2_v)
            sems = (sem, sem2)
            h0 = pltpu.async_copy(tab_s.at[sidx_v.at[g0]], buf_v, sem)
            for k in range(8):
                h0.wait()
                if k + 1 < 8:
                    h0 = pltpu.async_copy(tab_s.at[sidx_v.at[g0 + k + 1]],
                                          bufs[(k + 1) % 2], sems[(k + 1) % 2])
                pltpu.sync_copy(bufs[k % 2], acc_s.at[didx_v.at[g0 + k]],
                                add=True)

        plsc.subcore_barrier()
        pltpu.sync_copy(acc_s.at[pl.ds(row0, ROWS_PER_SUB)],
                        out_hbm.at[cid, h, pl.ds(row0, ROWS_PER_SUB)])


def _sc_agg(g2h, src2, dst2):
    mesh = plsc.VectorSubcoreMesh(core_axis_name="c", subcore_axis_name="s")
    f = pl.kernel(
        _sc_agg_body,
        out_type=jax.ShapeDtypeStruct((2, 2, NPAD, HALF), jnp.float32),
        mesh=mesh,
        compiler_params=pltpu.CompilerParams(use_tc_tiling_on_sc=False),
        scratch_types=[
            pltpu.VMEM((NCHUNK, CHUNK), jnp.int32),
            pltpu.VMEM((NCHUNK, CHUNK), jnp.int32),
            pltpu.VMEM((CHUNK, HALF), jnp.float32),
            pltpu.VMEM((CHUNK, HALF), jnp.float32),
            pltpu.VMEM_SHARED((NPAD, HALF), jnp.float32),
            pltpu.VMEM_SHARED((NPAD, HALF), jnp.float32),
            pltpu.SemaphoreType.DMA,
            pltpu.SemaphoreType.DMA,
        ],
    )
    return f(g2h, src2, dst2)


def _d_from_degp(degp_ref):
    deg = degp_ref[0, :, 0:1] + degp_ref[1, :, 0:1] + 1.0
    return lax.rsqrt(deg)


def _tc_first_body(x_ref, degp_ref, w_ref, o_ref):
    d = _d_from_degp(degp_ref)
    h = jnp.dot(x_ref[...], w_ref[...],
                precision=lax.Precision.HIGHEST,
                preferred_element_type=jnp.float32)
    g = d * h
    o_ref[0] = g[:, :HALF]
    o_ref[1] = g[:, HALF:]


BLK = 1024
NBLK = NPAD // BLK


def _tc_first(xp, degp, W1):
    return pl.pallas_call(
        _tc_first_body,
        grid=(NBLK,),
        in_specs=[
            pl.BlockSpec((BLK, D), lambda i: (i, 0)),
            pl.BlockSpec((2, BLK, DEG_W), lambda i: (0, i, 0)),
            pl.BlockSpec((D, D), lambda i: (0, 0)),
        ],
        out_specs=pl.BlockSpec((2, BLK, HALF), lambda i: (0, i, 0)),
        out_shape=jax.ShapeDtypeStruct((2, NPAD, HALF), jnp.float32),
    )(xp, degp, W1)


def _tc_mid_body(q_ref, g1_ref, degp_ref, b1_ref, w2_ref, o_ref):
    d = _d_from_degp(degp_ref)
    s_lo = q_ref[0, 0] + q_ref[1, 0] + g1_ref[0]
    s_hi = q_ref[0, 1] + q_ref[1, 1] + g1_ref[1]
    s = jnp.concatenate([s_lo, s_hi], axis=1)
    z = jnp.maximum(d * s + b1_ref[...], 0.0)
    h = jnp.dot(z, w2_ref[...],
                precision=lax.Precision.HIGHEST,
                preferred_element_type=jnp.float32)
    g = d * h
    o_ref[0] = g[:, :HALF]
    o_ref[1] = g[:, HALF:]


def _tc_mid(q, g1, degp, b1, W2):
    return pl.pallas_call(
        _tc_mid_body,
        grid=(NBLK,),
        in_specs=[
            pl.BlockSpec((2, 2, BLK, HALF), lambda i: (0, 0, i, 0)),
            pl.BlockSpec((2, BLK, HALF), lambda i: (0, i, 0)),
            pl.BlockSpec((2, BLK, DEG_W), lambda i: (0, i, 0)),
            pl.BlockSpec((1, D), lambda i: (0, 0)),
            pl.BlockSpec((D, D), lambda i: (0, 0)),
        ],
        out_specs=pl.BlockSpec((2, BLK, HALF), lambda i: (0, i, 0)),
        out_shape=jax.ShapeDtypeStruct((2, NPAD, HALF), jnp.float32),
    )(q, g1, degp, b1, W2)


def _tc_head_body(r_ref, g2_ref, degp_ref, b2_ref, wfc_ref, bfc_ref, o_ref):
    d = _d_from_degp(degp_ref)
    s_lo = r_ref[0, 0] + r_ref[1, 0] + g2_ref[0]
    s_hi = r_ref[0, 1] + r_ref[1, 1] + g2_ref[1]
    s = jnp.concatenate([s_lo, s_hi], axis=1)
    z2 = jnp.maximum(d * s + b2_ref[...], 0.0)
    h = jnp.dot(z2, wfc_ref[...],
                precision=lax.Precision.HIGHEST,
                preferred_element_type=jnp.float32)
    o_ref[...] = jax.nn.sigmoid(h + bfc_ref[...])


def _tc_head(r, g2, degp, b2, Wfcp, bfcp):
    return pl.pallas_call(
        _tc_head_body,
        grid=(NBLK,),
        in_specs=[
            pl.BlockSpec((2, 2, BLK, HALF), lambda i: (0, 0, i, 0)),
            pl.BlockSpec((2, BLK, HALF), lambda i: (0, i, 0)),
            pl.BlockSpec((2, BLK, DEG_W), lambda i: (0, i, 0)),
            pl.BlockSpec((1, D), lambda i: (0, 0)),
            pl.BlockSpec((D, 8), lambda i: (0, 0)),
            pl.BlockSpec((1, 8), lambda i: (0, 0)),
        ],
        out_specs=pl.BlockSpec((BLK, 8), lambda i: (i, 0)),
        out_shape=jax.ShapeDtypeStruct((NPAD, 8), jnp.float32),
    )(r, g2, degp, b2, Wfcp, bfcp)


def kernel(x, edge_index, W1, b1, W2, b2, Wfc, bfc):
    xp = jnp.concatenate([x, jnp.zeros((NPAD - N, D), x.dtype)], axis=0)
    pad_idx = jnp.full((EPAD - E,), N, jnp.int32)
    src_pad = jnp.concatenate([edge_index[0], pad_idx])
    dst_pad = jnp.concatenate([edge_index[1], pad_idx])
    src2 = src_pad.reshape(NW * NCHUNK, CHUNK)
    dst2 = dst_pad.reshape(NW * NCHUNK, CHUNK)

    degp = _sc_deg(dst2)
    g1 = _tc_first(xp, degp, W1)
    q = _sc_agg(g1, src2, dst2)
    g2 = _tc_mid(q, g1, degp, b1.reshape(1, D), W2)
    r = _sc_agg(g2, src2, dst2)
    Wfcp = jnp.pad(Wfc, ((0, 0), (0, 7)))
    bfcp = jnp.broadcast_to(bfc.reshape(1, 1), (1, 8))
    ph = _tc_head(r, g2, degp, b2.reshape(1, D), Wfcp, bfcp)
    return ph[:N, 0:1]


# speedup vs baseline: 2.6772x; 1.0285x over previous
import jax
import jax.numpy as jnp
from jax import lax
from jax.experimental import pallas as pl
from jax.experimental.pallas import tpu as pltpu
from jax.experimental.pallas import tpu_sc as plsc

N = 10000
E = 320000
D = 128
HALF = 64
NPAD = 10240
NW = 32
EPT = 10240
EPAD = NW * EPT
CHUNK = 128
NCHUNK = EPT // CHUNK
NSUB = 16
ROWS_PER_SUB = NPAD // NSUB
DEG_W = 16


def _sc_deg_body(dst_hbm, out_hbm, didx_v, ones_v, zb_v, acc_s):
    cid = lax.axis_index("c")
    sid = lax.axis_index("s")
    wid = sid * 2 + cid

    @pl.loop(0, CHUNK)
    def _(r):
        ones_v[r, :] = jnp.ones((DEG_W,), jnp.float32)
        zb_v[r, :] = jnp.zeros((DEG_W,), jnp.float32)

    row0 = sid * ROWS_PER_SUB
    for k in range(ROWS_PER_SUB // CHUNK):
        pltpu.sync_copy(zb_v, acc_s.at[pl.ds(row0 + k * CHUNK, CHUNK)])
    pltpu.sync_copy(dst_hbm.at[pl.ds(wid * NCHUNK, NCHUNK)], didx_v)
    plsc.subcore_barrier()

    @pl.loop(0, NCHUNK)
    def _(ci):
        pltpu.sync_copy(ones_v, acc_s.at[didx_v.at[ci]], add=True)

    plsc.subcore_barrier()
    pltpu.sync_copy(acc_s.at[pl.ds(row0, ROWS_PER_SUB)],
                    out_hbm.at[cid, pl.ds(row0, ROWS_PER_SUB)])


def _sc_deg(dst2):
    mesh = plsc.VectorSubcoreMesh(core_axis_name="c", subcore_axis_name="s")
    f = pl.kernel(
        _sc_deg_body,
        out_type=jax.ShapeDtypeStruct((2, NPAD, DEG_W), jnp.float32),
        mesh=mesh,
        scratch_types=[
            pltpu.VMEM((NCHUNK, CHUNK), jnp.int32),
            pltpu.VMEM((CHUNK, DEG_W), jnp.float32),
            pltpu.VMEM((CHUNK, DEG_W), jnp.float32),
            pltpu.VMEM_SHARED((NPAD, DEG_W), jnp.float32),
        ],
    )
    return f(dst2)


def _zero_buf(buf_v):
    @pl.loop(0, CHUNK)
    def _(r):
        for c in range(HALF // 16):
            buf_v[r, pl.ds(c * 16, 16)] = jnp.zeros((16,), jnp.float32)


def _sc_agg_body(g2h_hbm, src_hbm, dst_hbm, out_hbm, sidx_v, didx_v, buf_v,
                 buf2_v, tab_s, acc_s, sem, sem2):
    cid = lax.axis_index("c")
    sid = lax.axis_index("s")
    wid = sid * 2 + cid
    row0 = sid * ROWS_PER_SUB

    pltpu.sync_copy(src_hbm.at[pl.ds(wid * NCHUNK, NCHUNK)], sidx_v)
    pltpu.sync_copy(dst_hbm.at[pl.ds(wid * NCHUNK, NCHUNK)], didx_v)

    for h in range(2):
        _zero_buf(buf_v)
        pltpu.sync_copy(g2h_hbm.at[h, pl.ds(row0, ROWS_PER_SUB)],
                        tab_s.at[pl.ds(row0, ROWS_PER_SUB)])
        for k in range(ROWS_PER_SUB // CHUNK):
            pltpu.sync_copy(buf_v, acc_s.at[pl.ds(row0 + k * CHUNK, CHUNK)])
        plsc.subcore_barrier()

        @pl.loop(0, NCHUNK, step=16)
        def _(g0):
            bufs = (buf_v, buf2_v)
            sems = (sem, sem2)
            h0 = pltpu.async_copy(tab_s.at[sidx_v.at[g0]], buf_v, sem)
            for k in range(16):
                h0.wait()
                if k + 1 < 16:
                    h0 = pltpu.async_copy(tab_s.at[sidx_v.at[g0 + k + 1]],
                                          bufs[(k + 1) % 2], sems[(k + 1) % 2])
                pltpu.sync_copy(bufs[k % 2], acc_s.at[didx_v.at[g0 + k]],
                                add=True)

        plsc.subcore_barrier()
        pltpu.sync_copy(acc_s.at[pl.ds(row0, ROWS_PER_SUB)],
                        out_hbm.at[cid, h, pl.ds(row0, ROWS_PER_SUB)])


def _sc_agg(g2h, src2, dst2):
    mesh = plsc.VectorSubcoreMesh(core_axis_name="c", subcore_axis_name="s")
    f = pl.kernel(
        _sc_agg_body,
        out_type=jax.ShapeDtypeStruct((2, 2, NPAD, HALF), jnp.float32),
        mesh=mesh,
        compiler_params=pltpu.CompilerParams(use_tc_tiling_on_sc=False),
        scratch_types=[
            pltpu.VMEM((NCHUNK, CHUNK), jnp.int32),
            pltpu.VMEM((NCHUNK, CHUNK), jnp.int32),
            pltpu.VMEM((CHUNK, HALF), jnp.float32),
            pltpu.VMEM((CHUNK, HALF), jnp.float32),
            pltpu.VMEM_SHARED((NPAD, HALF), jnp.float32),
            pltpu.VMEM_SHARED((NPAD, HALF), jnp.float32),
            pltpu.SemaphoreType.DMA,
            pltpu.SemaphoreType.DMA,
        ],
    )
    return f(g2h, src2, dst2)


def _d_from_degp(degp_ref):
    deg = degp_ref[0, :, 0:1] + degp_ref[1, :, 0:1] + 1.0
    return lax.rsqrt(deg)


def _tc_first_body(x_ref, degp_ref, w_ref, o_ref):
    d = _d_from_degp(degp_ref)
    h = jnp.dot(x_ref[...], w_ref[...],
                precision=lax.Precision.HIGHEST,
                preferred_element_type=jnp.float32)
    g = d * h
    o_ref[0] = g[:, :HALF]
    o_ref[1] = g[:, HALF:]


BLK = 1024
NBLK = NPAD // BLK


def _tc_first(xp, degp, W1):
    return pl.pallas_call(
        _tc_first_body,
        grid=(NBLK,),
        in_specs=[
            pl.BlockSpec((BLK, D), lambda i: (i, 0)),
            pl.BlockSpec((2, BLK, DEG_W), lambda i: (0, i, 0)),
            pl.BlockSpec((D, D), lambda i: (0, 0)),
        ],
        out_specs=pl.BlockSpec((2, BLK, HALF), lambda i: (0, i, 0)),
        out_shape=jax.ShapeDtypeStruct((2, NPAD, HALF), jnp.float32),
    )(xp, degp, W1)


def _tc_mid_body(q_ref, g1_ref, degp_ref, b1_ref, w2_ref, o_ref):
    d = _d_from_degp(degp_ref)
    s_lo = q_ref[0, 0] + q_ref[1, 0] + g1_ref[0]
    s_hi = q_ref[0, 1] + q_ref[1, 1] + g1_ref[1]
    s = jnp.concatenate([s_lo, s_hi], axis=1)
    z = jnp.maximum(d * s + b1_ref[...], 0.0)
    h = jnp.dot(z, w2_ref[...],
                precision=lax.Precision.HIGHEST,
                preferred_element_type=jnp.float32)
    g = d * h
    o_ref[0] = g[:, :HALF]
    o_ref[1] = g[:, HALF:]


def _tc_mid(q, g1, degp, b1, W2):
    return pl.pallas_call(
        _tc_mid_body,
        grid=(NBLK,),
        in_specs=[
            pl.BlockSpec((2, 2, BLK, HALF), lambda i: (0, 0, i, 0)),
            pl.BlockSpec((2, BLK, HALF), lambda i: (0, i, 0)),
            pl.BlockSpec((2, BLK, DEG_W), lambda i: (0, i, 0)),
            pl.BlockSpec((1, D), lambda i: (0, 0)),
            pl.BlockSpec((D, D), lambda i: (0, 0)),
        ],
        out_specs=pl.BlockSpec((2, BLK, HALF), lambda i: (0, i, 0)),
        out_shape=jax.ShapeDtypeStruct((2, NPAD, HALF), jnp.float32),
    )(q, g1, degp, b1, W2)


def _tc_head_body(r_ref, g2_ref, degp_ref, b2_ref, wfc_ref, bfc_ref, o_ref):
    d = _d_from_degp(degp_ref)
    s_lo = r_ref[0, 0] + r_ref[1, 0] + g2_ref[0]
    s_hi = r_ref[0, 1] + r_ref[1, 1] + g2_ref[1]
    s = jnp.concatenate([s_lo, s_hi], axis=1)
    z2 = jnp.maximum(d * s + b2_ref[...], 0.0)
    h = jnp.dot(z2, wfc_ref[...],
                precision=lax.Precision.HIGHEST,
                preferred_element_type=jnp.float32)
    o_ref[...] = jax.nn.sigmoid(h + bfc_ref[...])


def _tc_head(r, g2, degp, b2, Wfcp, bfcp):
    return pl.pallas_call(
        _tc_head_body,
        grid=(NBLK,),
        in_specs=[
            pl.BlockSpec((2, 2, BLK, HALF), lambda i: (0, 0, i, 0)),
            pl.BlockSpec((2, BLK, HALF), lambda i: (0, i, 0)),
            pl.BlockSpec((2, BLK, DEG_W), lambda i: (0, i, 0)),
            pl.BlockSpec((1, D), lambda i: (0, 0)),
            pl.BlockSpec((D, 8), lambda i: (0, 0)),
            pl.BlockSpec((1, 8), lambda i: (0, 0)),
        ],
        out_specs=pl.BlockSpec((BLK, 8), lambda i: (i, 0)),
        out_shape=jax.ShapeDtypeStruct((NPAD, 8), jnp.float32),
    )(r, g2, degp, b2, Wfcp, bfcp)


def kernel(x, edge_index, W1, b1, W2, b2, Wfc, bfc):
    xp = jnp.concatenate([x, jnp.zeros((NPAD - N, D), x.dtype)], axis=0)
    pad_idx = jnp.full((EPAD - E,), N, jnp.int32)
    src_pad = jnp.concatenate([edge_index[0], pad_idx])
    dst_pad = jnp.concatenate([edge_index[1], pad_idx])
    src2 = src_pad.reshape(NW * NCHUNK, CHUNK)
    dst2 = dst_pad.reshape(NW * NCHUNK, CHUNK)

    degp = _sc_deg(dst2)
    g1 = _tc_first(xp, degp, W1)
    q = _sc_agg(g1, src2, dst2)
    g2 = _tc_mid(q, g1, degp, b1.reshape(1, D), W2)
    r = _sc_agg(g2, src2, dst2)
    Wfcp = jnp.pad(Wfc, ((0, 0), (0, 7)))
    bfcp = jnp.broadcast_to(bfc.reshape(1, 1), (1, 8))
    ph = _tc_head(r, g2, degp, b2.reshape(1, D), Wfcp, bfcp)
    return ph[:N, 0:1]
